# Initial kernel scaffold; baseline (speedup 1.0000x reference)
import jax
import jax.numpy as jnp
from jax.experimental import pallas as pl


def kernel(x, self_attn_input, halt_mask, layer_idx, encoder_padding_mask, ln1_g, ln1_b, ln2_g, ln2_b, Wq, bq, Wk, bk, Wv, bv, Wo, bo, Wg, W1, b1, W2, b2):
    raise NotImplementedError("write your pallas kernel here")


# speedup vs baseline: 1.0296x; 1.0296x over previous
import functools

import jax
import jax.numpy as jnp
from jax import lax
from jax.experimental import pallas as pl
from jax.experimental.pallas import tpu as pltpu
from jax.experimental.pallas import tpu_sc as plsc

SEQ, B, EMBED, HEADS, HDIM = 2048, 1, 1024, 16, 64
E, K, FF = 8, 2, 2048
N = SEQ * B

BM = 256
BQ = 256
NPAD = K * N + E * BM
NB = NPAD // BM

_f32 = jnp.float32


def _ln(x, g, b):
    m = jnp.mean(x, axis=-1, keepdims=True)
    v = jnp.mean((x - m) ** 2, axis=-1, keepdims=True)
    return (x - m) * lax.rsqrt(v + 1e-5) * g + b


def _qkv_body(x_ref, sa_ref, g_ref, b_ref, wq_ref, bq_ref, wk_ref, bk_ref,
              wv_ref, bv_ref, q_ref, k_ref, v_ref):
    xq = _ln(x_ref[...], g_ref[...], b_ref[...])
    xkv = _ln(sa_ref[...], g_ref[...], b_ref[...])
    q_ref[...] = jnp.dot(xq, wq_ref[...], preferred_element_type=_f32) + bq_ref[...]
    k_ref[...] = jnp.dot(xkv, wk_ref[...], preferred_element_type=_f32) + bk_ref[...]
    v_ref[...] = jnp.dot(xkv, wv_ref[...], preferred_element_type=_f32) + bv_ref[...]


def _attn_body(q_ref, k_ref, v_ref, pm_ref, o_ref):
    q = q_ref[0]
    k = k_ref[0]
    v = v_ref[0]
    s = lax.dot_general(q, k, (((1,), (1,)), ((), ())),
                        preferred_element_type=_f32) * 0.125
    s = s + pm_ref[...]
    m = jnp.max(s, axis=-1, keepdims=True)
    p = jnp.exp(s - m)
    o = jnp.dot(p, v, preferred_element_type=_f32)
    o_ref[0] = o / jnp.sum(p, axis=-1, keepdims=True)


def _oproj_body(a_ref, wo_ref, bo_ref, hm_ref, x_ref, o_ref):
    y = jnp.dot(a_ref[...], wo_ref[...], preferred_element_type=_f32) + bo_ref[...]
    o_ref[...] = x_ref[...] + y * hm_ref[...]


def _router_body(x_ref, g_ref, b_ref, wg_ref, hm_ref,
                 xn_ref, i1_ref, i2_ref, g1_ref, g2_ref):
    xn = _ln(x_ref[...], g_ref[...], b_ref[...])
    xn_ref[...] = xn
    lg = jnp.dot(xn, wg_ref[...], preferred_element_type=_f32)
    idx = lax.broadcasted_iota(jnp.int32, lg.shape, 1)
    m1 = jnp.max(lg, axis=-1, keepdims=True)
    i1 = jnp.min(jnp.where(lg == m1, idx, E), axis=-1, keepdims=True)
    lg2 = jnp.where(idx == i1, -jnp.inf, lg)
    m2 = jnp.max(lg2, axis=-1, keepdims=True)
    i2 = jnp.min(jnp.where(lg2 == m2, idx, E), axis=-1, keepdims=True)
    g1 = 1.0 / (1.0 + jnp.exp(m2 - m1))
    hm = hm_ref[...]
    i1_ref[...] = i1
    i2_ref[...] = i2
    g1_ref[...] = g1 * hm
    g2_ref[...] = (1.0 - g1) * hm


def _moe_body(eid_ref, xs_ref, w1_ref, b1_ref, w2_ref, b2_ref, gt_ref, ys_ref):
    bidx = pl.program_id(0)

    @pl.when(eid_ref[bidx] < E)
    def _():
        h = jnp.maximum(
            jnp.dot(xs_ref[...], w1_ref[0], preferred_element_type=_f32)
            + b1_ref[0], 0.0)
        y = jnp.dot(h, w2_ref[0], preferred_element_type=_f32) + b2_ref[0]
        ys_ref[...] = y * gt_ref[...]


def _fin_body(x_ref, y1_ref, y2_ref, o_ref):
    o_ref[...] = x_ref[...] + y1_ref[...] + y2_ref[...]


_SC_CORES, _SC_SUBCORES = 2, 16
_NW = _SC_CORES * _SC_SUBCORES
_CHUNK = 64


def _sc_gather(table, idx):
    rows = idx.shape[0]
    rpw = rows // _NW
    nch = rpw // _CHUNK
    mesh = plsc.VectorSubcoreMesh(core_axis_name="c", subcore_axis_name="s")

    @functools.partial(
        pl.kernel, mesh=mesh,
        out_type=jax.ShapeDtypeStruct((rows, table.shape[1]), _f32),
        scratch_types=[
            pltpu.VMEM((_CHUNK,), jnp.int32),
            pltpu.VMEM((_CHUNK, table.shape[1]), _f32),
            pltpu.SemaphoreType.DMA,
        ],
    )
    def k(table_hbm, idx_hbm, out_hbm, idx_v, rows_v, sem):
        wid = lax.axis_index("s") * _SC_CORES + lax.axis_index("c")
        base = wid * rpw

        def body(ci, carry):
            o = base + ci * _CHUNK
            pltpu.sync_copy(idx_hbm.at[pl.ds(o, _CHUNK)], idx_v)
            pltpu.async_copy(table_hbm.at[idx_v], rows_v, sem).wait()
            pltpu.sync_copy(rows_v, out_hbm.at[pl.ds(o, _CHUNK)])
            return carry

        lax.fori_loop(0, nch, body, 0)

    return k(table, idx)


def _route_metadata(i1, i2, g1, g2):
    ei = jnp.stack([i1, i2], axis=1).reshape(-1)
    gi = jnp.stack([g1, g2], axis=1).reshape(-1)
    tok = (jnp.arange(K * N, dtype=jnp.int32) // K)
    oh = ei[:, None] == jnp.arange(E, dtype=jnp.int32)[None, :]
    counts = jnp.sum(oh.astype(jnp.int32), axis=0)
    padded = ((counts + BM - 1) // BM) * BM
    ends = jnp.cumsum(padded)
    offsets = ends - padded
    rank = jnp.cumsum(oh.astype(jnp.int32), axis=0) - 1
    rankj = jnp.take_along_axis(rank, ei[:, None], axis=1)[:, 0]
    pos = offsets[ei] + rankj
    token_map = jnp.zeros((NPAD,), jnp.int32).at[pos].set(tok)
    gate_map = jnp.zeros((NPAD,), _f32).at[pos].set(gi)
    bstart = jnp.arange(NB, dtype=jnp.int32) * BM
    beid = jnp.searchsorted(ends, bstart, side="right").astype(jnp.int32)
    beid = jnp.where(bstart < ends[-1], beid, E).astype(jnp.int32)
    pos2 = pos.reshape(N, K)
    return token_map, gate_map, beid, pos2


def kernel(x, self_attn_input, halt_mask, layer_idx, encoder_padding_mask,
           ln1_g, ln1_b, ln2_g, ln2_b, Wq, bq, Wk, bk, Wv, bv, Wo, bo,
           Wg, W1, b1, W2, b2):
    x2 = x.reshape(N, EMBED)
    sa2 = self_attn_input.reshape(N, EMBED)
    hm = (~halt_mask.reshape(-1))[:, None].astype(_f32)
    pm = jnp.where(encoder_padding_mask.reshape(1, -1), -1e8, 0.0).astype(_f32)

    g1v = ln1_g.reshape(1, EMBED)
    b1v = ln1_b.reshape(1, EMBED)
    g2v = ln2_g.reshape(1, EMBED)
    b2v = ln2_b.reshape(1, EMBED)
    bqv = bq.reshape(1, EMBED)
    bkv = bk.reshape(1, EMBED)
    bvv = bv.reshape(1, EMBED)
    bov = bo.reshape(1, EMBED)

    row_spec = pl.BlockSpec((BM, EMBED), lambda i: (i, 0))
    w_spec = pl.BlockSpec((EMBED, EMBED), lambda i: (0, 0))
    vec_spec = pl.BlockSpec((1, EMBED), lambda i: (0, 0))
    col1_spec = pl.BlockSpec((BM, 1), lambda i: (i, 0))

    q, kk, v = pl.pallas_call(
        _qkv_body,
        grid=(N // BM,),
        in_specs=[row_spec, row_spec, vec_spec, vec_spec,
                  w_spec, vec_spec, w_spec, vec_spec, w_spec, vec_spec],
        out_specs=[row_spec, row_spec, row_spec],
        out_shape=[jax.ShapeDtypeStruct((N, EMBED), _f32)] * 3,
    )(x2, sa2, g1v, b1v, Wq, bqv, Wk, bkv, Wv, bvv)

    q3 = q.reshape(N, HEADS, HDIM).transpose(1, 0, 2)
    k3 = kk.reshape(N, HEADS, HDIM).transpose(1, 0, 2)
    v3 = v.reshape(N, HEADS, HDIM).transpose(1, 0, 2)

    qb_spec = pl.BlockSpec((1, BQ, HDIM), lambda h, i: (h, i, 0))
    kv_spec = pl.BlockSpec((1, N, HDIM), lambda h, i: (h, 0, 0))
    pm_spec = pl.BlockSpec((1, N), lambda h, i: (0, 0))
    attn3 = pl.pallas_call(
        _attn_body,
        grid=(HEADS, N // BQ),
        in_specs=[qb_spec, kv_spec, kv_spec, pm_spec],
        out_specs=qb_spec,
        out_shape=jax.ShapeDtypeStruct((HEADS, N, HDIM), _f32),
    )(q3, k3, v3, pm)
    attn = attn3.transpose(1, 0, 2).reshape(N, EMBED)

    x1 = pl.pallas_call(
        _oproj_body,
        grid=(N // BM,),
        in_specs=[row_spec, w_spec, vec_spec, col1_spec, row_spec],
        out_specs=row_spec,
        out_shape=jax.ShapeDtypeStruct((N, EMBED), _f32),
    )(attn, Wo, bov, hm, x2)

    wg_spec = pl.BlockSpec((EMBED, E), lambda i: (0, 0))
    i1_spec = pl.BlockSpec((BM, 1), lambda i: (i, 0))
    xn, i1, i2, gg1, gg2 = pl.pallas_call(
        _router_body,
        grid=(N // BM,),
        in_specs=[row_spec, vec_spec, vec_spec, wg_spec, col1_spec],
        out_specs=[row_spec, i1_spec, i1_spec, i1_spec, i1_spec],
        out_shape=[
            jax.ShapeDtypeStruct((N, EMBED), _f32),
            jax.ShapeDtypeStruct((N, 1), jnp.int32),
            jax.ShapeDtypeStruct((N, 1), jnp.int32),
            jax.ShapeDtypeStruct((N, 1), _f32),
            jax.ShapeDtypeStruct((N, 1), _f32),
        ],
    )(x1, g2v, b2v, Wg, hm)

    token_map, gate_map, beid, pos = _route_metadata(
        i1.reshape(-1), i2.reshape(-1), gg1.reshape(-1), gg2.reshape(-1))

    xs = _sc_gather(xn, token_map)

    def _wsel(b, eid):
        return jnp.minimum(eid[b], E - 1)

    grid_spec = pltpu.PrefetchScalarGridSpec(
        num_scalar_prefetch=1,
        grid=(NB,),
        in_specs=[
            pl.BlockSpec((BM, EMBED), lambda b, eid: (b, 0)),
            pl.BlockSpec((1, EMBED, FF), lambda b, eid: (_wsel(b, eid), 0, 0)),
            pl.BlockSpec((1, 1, FF), lambda b, eid: (_wsel(b, eid), 0, 0)),
            pl.BlockSpec((1, FF, EMBED), lambda b, eid: (_wsel(b, eid), 0, 0)),
            pl.BlockSpec((1, 1, EMBED), lambda b, eid: (_wsel(b, eid), 0, 0)),
            pl.BlockSpec((BM, 1), lambda b, eid: (b, 0)),
        ],
        out_specs=pl.BlockSpec((BM, EMBED), lambda b, eid: (b, 0)),
    )
    ys = pl.pallas_call(
        _moe_body,
        grid_spec=grid_spec,
        out_shape=jax.ShapeDtypeStruct((NPAD, EMBED), _f32),
    )(beid, xs, W1, b1.reshape(E, 1, FF), W2, b2.reshape(E, 1, EMBED),
      gate_map[:, None])

    pos_all = pos.T.reshape(-1)
    y12 = _sc_gather(ys, pos_all)
    y1 = y12[:N]
    y2 = y12[N:]

    out = pl.pallas_call(
        _fin_body,
        grid=(N // BM,),
        in_specs=[row_spec, row_spec, row_spec],
        out_specs=row_spec,
        out_shape=jax.ShapeDtypeStruct((N, EMBED), _f32),
    )(x1, y1, y2)

    return out.reshape(SEQ, B, EMBED), jnp.float32(0.0)


# speedup vs baseline: 1.3057x; 1.2681x over previous
import functools

import jax
import jax.numpy as jnp
from jax import lax
from jax.experimental import pallas as pl
from jax.experimental.pallas import tpu as pltpu
from jax.experimental.pallas import tpu_sc as plsc

SEQ, B, EMBED, HEADS, HDIM = 2048, 1, 1024, 16, 64
E, K, FF = 8, 2, 2048
N = SEQ * B

BM = 256
BQ = 256
NPAD = K * N + E * BM
NB = NPAD // BM

_f32 = jnp.float32


def _ln(x, g, b):
    m = jnp.mean(x, axis=-1, keepdims=True)
    v = jnp.mean((x - m) ** 2, axis=-1, keepdims=True)
    return (x - m) * lax.rsqrt(v + 1e-5) * g + b


def _qkv_body(x_ref, sa_ref, g_ref, b_ref, wq_ref, bq_ref, wk_ref, bk_ref,
              wv_ref, bv_ref, q_ref, k_ref, v_ref):
    xq = _ln(x_ref[...], g_ref[...], b_ref[...])
    xkv = _ln(sa_ref[...], g_ref[...], b_ref[...])
    q_ref[...] = jnp.dot(xq, wq_ref[...], preferred_element_type=_f32) + bq_ref[...]
    k_ref[...] = jnp.dot(xkv, wk_ref[...], preferred_element_type=_f32) + bk_ref[...]
    v_ref[...] = jnp.dot(xkv, wv_ref[...], preferred_element_type=_f32) + bv_ref[...]


def _attn_body(q_ref, k_ref, v_ref, pm_ref, o_ref):
    q = q_ref[0]
    k = k_ref[0]
    v = v_ref[0]
    s = lax.dot_general(q, k, (((1,), (1,)), ((), ())),
                        preferred_element_type=_f32) * 0.125
    s = s + pm_ref[...]
    m = jnp.max(s, axis=-1, keepdims=True)
    p = jnp.exp(s - m)
    o = jnp.dot(p, v, preferred_element_type=_f32)
    o_ref[0] = o / jnp.sum(p, axis=-1, keepdims=True)


def _oproj_router_body(a_ref, wo_ref, bo_ref, hm_ref, x_ref, g2_ref, b2_ref,
                       wg_ref, x1_ref, xn_ref, i1_ref, i2_ref, g1_ref, gg2_ref):
    y = jnp.dot(a_ref[...], wo_ref[...], preferred_element_type=_f32) + bo_ref[...]
    hm = hm_ref[...]
    x1 = x_ref[...] + y * hm
    x1_ref[...] = x1
    xn = _ln(x1, g2_ref[...], b2_ref[...])
    xn_ref[...] = xn
    lg = jnp.dot(xn, wg_ref[...], preferred_element_type=_f32)
    idx = lax.broadcasted_iota(jnp.int32, lg.shape, 1)
    m1 = jnp.max(lg, axis=-1, keepdims=True)
    i1 = jnp.min(jnp.where(lg == m1, idx, E), axis=-1, keepdims=True)
    lg2 = jnp.where(idx == i1, -jnp.inf, lg)
    m2 = jnp.max(lg2, axis=-1, keepdims=True)
    i2 = jnp.min(jnp.where(lg2 == m2, idx, E), axis=-1, keepdims=True)
    g1 = 1.0 / (1.0 + jnp.exp(m2 - m1))
    i1_ref[...] = i1
    i2_ref[...] = i2
    g1_ref[...] = g1 * hm
    gg2_ref[...] = (1.0 - g1) * hm


def _moe_body(eid_ref, xs_ref, w1_ref, b1_ref, w2_ref, b2_ref, ys_ref):
    bidx = pl.program_id(0)

    @pl.when(eid_ref[bidx] < E)
    def _():
        h = jnp.maximum(
            jnp.dot(xs_ref[...], w1_ref[0], preferred_element_type=_f32)
            + b1_ref[0], 0.0)
        ys_ref[...] = jnp.dot(h, w2_ref[0], preferred_element_type=_f32) + b2_ref[0]


def _fin_body(x_ref, y1_ref, y2_ref, g1_ref, g2_ref, o_ref):
    o_ref[...] = (x_ref[...] + g1_ref[...] * y1_ref[...]
                  + g2_ref[...] * y2_ref[...])


_SC_CORES, _SC_SUBCORES = 2, 16
_NW = _SC_CORES * _SC_SUBCORES
_CH = 32


def _sc_dispatch(xn, tok, pos):
    M = tok.shape[0]
    mpw = M // _NW
    nch = mpw // _CH
    mesh = plsc.VectorSubcoreMesh(core_axis_name="c", subcore_axis_name="s")

    @functools.partial(
        pl.kernel, mesh=mesh,
        out_type=jax.ShapeDtypeStruct((NPAD, EMBED), _f32),
        scratch_types=[
            pltpu.VMEM((2, _CH), jnp.int32),
            pltpu.VMEM((2, _CH), jnp.int32),
            pltpu.VMEM((2, _CH, EMBED), _f32),
            pltpu.SemaphoreType.DMA,
            pltpu.SemaphoreType.DMA,
            pltpu.SemaphoreType.DMA,
        ],
    )
    def k(xn_hbm, tok_hbm, pos_hbm, out_hbm, tok_v, pos_v, rows_v,
          gsem, ssem0, ssem1):
        wid = lax.axis_index("s") * _SC_CORES + lax.axis_index("c")
        base = wid * mpw
        ssems = (ssem0, ssem1)
        pend = [None, None]
        for c in range(nch):
            b = c % 2
            o = base + c * _CH
            if pend[b] is not None:
                pend[b].wait()
            pltpu.sync_copy(tok_hbm.at[pl.ds(o, _CH)], tok_v.at[b])
            pltpu.sync_copy(pos_hbm.at[pl.ds(o, _CH)], pos_v.at[b])
            pltpu.async_copy(xn_hbm.at[tok_v.at[b]], rows_v.at[b], gsem).wait()
            pend[b] = pltpu.async_copy(rows_v.at[b], out_hbm.at[pos_v.at[b]],
                                       ssems[b])
        for p in pend:
            if p is not None:
                p.wait()

    return k(xn, tok, pos)


def _sc_gather(table, idx):
    rows = idx.shape[0]
    rpw = rows // _NW
    nch = rpw // _CH
    mesh = plsc.VectorSubcoreMesh(core_axis_name="c", subcore_axis_name="s")

    @functools.partial(
        pl.kernel, mesh=mesh,
        out_type=jax.ShapeDtypeStruct((rows, table.shape[1]), _f32),
        scratch_types=[
            pltpu.VMEM((2, _CH), jnp.int32),
            pltpu.VMEM((2, _CH, table.shape[1]), _f32),
            pltpu.SemaphoreType.DMA,
            pltpu.SemaphoreType.DMA,
            pltpu.SemaphoreType.DMA,
        ],
    )
    def k(table_hbm, idx_hbm, out_hbm, idx_v, rows_v, gsem, ssem0, ssem1):
        wid = lax.axis_index("s") * _SC_CORES + lax.axis_index("c")
        base = wid * rpw
        ssems = (ssem0, ssem1)
        pend = [None, None]
        for c in range(nch):
            b = c % 2
            o = base + c * _CH
            if pend[b] is not None:
                pend[b].wait()
            pltpu.sync_copy(idx_hbm.at[pl.ds(o, _CH)], idx_v.at[b])
            pltpu.async_copy(table_hbm.at[idx_v.at[b]], rows_v.at[b],
                             gsem).wait()
            pend[b] = pltpu.async_copy(rows_v.at[b],
                                       out_hbm.at[pl.ds(o, _CH)], ssems[b])
        for p in pend:
            if p is not None:
                p.wait()

    return k(table, idx)


def _route_metadata(i1, i2):
    ei = jnp.stack([i1, i2], axis=1).reshape(-1)
    tok = (jnp.arange(K * N, dtype=jnp.int32) // K)
    oh = (ei[:, None] == jnp.arange(E, dtype=jnp.int32)[None, :]).astype(jnp.int32)
    counts = jnp.sum(oh, axis=0)
    padded = ((counts + BM - 1) // BM) * BM
    ends = jnp.cumsum(padded)
    offsets = ends - padded
    csum = jnp.cumsum(oh, axis=0)
    rankj = jnp.sum((csum - 1) * oh, axis=1)
    posj = jnp.sum(offsets[None, :] * oh, axis=1) + rankj
    posj = posj.astype(jnp.int32)
    bstart = jnp.arange(NB, dtype=jnp.int32) * BM
    beid = jnp.sum((bstart[:, None] >= ends[None, :]).astype(jnp.int32), axis=1)
    beid = jnp.where(bstart < ends[-1], beid, E).astype(jnp.int32)
    return tok, posj, beid


def kernel(x, self_attn_input, halt_mask, layer_idx, encoder_padding_mask,
           ln1_g, ln1_b, ln2_g, ln2_b, Wq, bq, Wk, bk, Wv, bv, Wo, bo,
           Wg, W1, b1, W2, b2):
    x2 = x.reshape(N, EMBED)
    sa2 = self_attn_input.reshape(N, EMBED)
    hm = (~halt_mask.reshape(-1))[:, None].astype(_f32)
    pm = jnp.where(encoder_padding_mask.reshape(1, -1), -1e8, 0.0).astype(_f32)

    g1v = ln1_g.reshape(1, EMBED)
    b1v = ln1_b.reshape(1, EMBED)
    g2v = ln2_g.reshape(1, EMBED)
    b2v = ln2_b.reshape(1, EMBED)
    bqv = bq.reshape(1, EMBED)
    bkv = bk.reshape(1, EMBED)
    bvv = bv.reshape(1, EMBED)
    bov = bo.reshape(1, EMBED)

    row_spec = pl.BlockSpec((BM, EMBED), lambda i: (i, 0))
    w_spec = pl.BlockSpec((EMBED, EMBED), lambda i: (0, 0))
    vec_spec = pl.BlockSpec((1, EMBED), lambda i: (0, 0))
    col1_spec = pl.BlockSpec((BM, 1), lambda i: (i, 0))

    q, kk, v = pl.pallas_call(
        _qkv_body,
        grid=(N // BM,),
        in_specs=[row_spec, row_spec, vec_spec, vec_spec,
                  w_spec, vec_spec, w_spec, vec_spec, w_spec, vec_spec],
        out_specs=[row_spec, row_spec, row_spec],
        out_shape=[jax.ShapeDtypeStruct((N, EMBED), _f32)] * 3,
    )(x2, sa2, g1v, b1v, Wq, bqv, Wk, bkv, Wv, bvv)

    q3 = q.reshape(N, HEADS, HDIM).transpose(1, 0, 2)
    k3 = kk.reshape(N, HEADS, HDIM).transpose(1, 0, 2)
    v3 = v.reshape(N, HEADS, HDIM).transpose(1, 0, 2)

    qb_spec = pl.BlockSpec((1, BQ, HDIM), lambda h, i: (h, i, 0))
    kv_spec = pl.BlockSpec((1, N, HDIM), lambda h, i: (h, 0, 0))
    pm_spec = pl.BlockSpec((1, N), lambda h, i: (0, 0))
    attn3 = pl.pallas_call(
        _attn_body,
        grid=(HEADS, N // BQ),
        in_specs=[qb_spec, kv_spec, kv_spec, pm_spec],
        out_specs=qb_spec,
        out_shape=jax.ShapeDtypeStruct((HEADS, N, HDIM), _f32),
    )(q3, k3, v3, pm)
    attn = attn3.transpose(1, 0, 2).reshape(N, EMBED)

    wg_spec = pl.BlockSpec((EMBED, E), lambda i: (0, 0))
    i1_spec = pl.BlockSpec((BM, 1), lambda i: (i, 0))
    x1, xn, i1, i2, gg1, gg2 = pl.pallas_call(
        _oproj_router_body,
        grid=(N // BM,),
        in_specs=[row_spec, w_spec, vec_spec, col1_spec, row_spec,
                  vec_spec, vec_spec, wg_spec],
        out_specs=[row_spec, row_spec, i1_spec, i1_spec, i1_spec, i1_spec],
        out_shape=[
            jax.ShapeDtypeStruct((N, EMBED), _f32),
            jax.ShapeDtypeStruct((N, EMBED), _f32),
            jax.ShapeDtypeStruct((N, 1), jnp.int32),
            jax.ShapeDtypeStruct((N, 1), jnp.int32),
            jax.ShapeDtypeStruct((N, 1), _f32),
            jax.ShapeDtypeStruct((N, 1), _f32),
        ],
    )(attn, Wo, bov, hm, x2, g2v, b2v, Wg)

    tok, pos, beid = _route_metadata(i1.reshape(-1), i2.reshape(-1))

    xs = _sc_dispatch(xn, tok, pos)

    def _wsel(b, eid):
        return jnp.minimum(eid[b], E - 1)

    grid_spec = pltpu.PrefetchScalarGridSpec(
        num_scalar_prefetch=1,
        grid=(NB,),
        in_specs=[
            pl.BlockSpec((BM, EMBED), lambda b, eid: (b, 0)),
            pl.BlockSpec((1, EMBED, FF), lambda b, eid: (_wsel(b, eid), 0, 0)),
            pl.BlockSpec((1, 1, FF), lambda b, eid: (_wsel(b, eid), 0, 0)),
            pl.BlockSpec((1, FF, EMBED), lambda b, eid: (_wsel(b, eid), 0, 0)),
            pl.BlockSpec((1, 1, EMBED), lambda b, eid: (_wsel(b, eid), 0, 0)),
        ],
        out_specs=pl.BlockSpec((BM, EMBED), lambda b, eid: (b, 0)),
    )
    ys = pl.pallas_call(
        _moe_body,
        grid_spec=grid_spec,
        out_shape=jax.ShapeDtypeStruct((NPAD, EMBED), _f32),
    )(beid, xs, W1, b1.reshape(E, 1, FF), W2, b2.reshape(E, 1, EMBED))

    pos2 = pos.reshape(N, K)
    pos_all = pos2.T.reshape(-1)
    y12 = _sc_gather(ys, pos_all)
    y1 = y12[:N]
    y2 = y12[N:]

    out = pl.pallas_call(
        _fin_body,
        grid=(N // BM,),
        in_specs=[row_spec, row_spec, row_spec, col1_spec, col1_spec],
        out_specs=row_spec,
        out_shape=jax.ShapeDtypeStruct((N, EMBED), _f32),
    )(x1, y1, y2, gg1, gg2)

    return out.reshape(SEQ, B, EMBED), jnp.float32(0.0)


# speedup vs baseline: 1.4680x; 1.1243x over previous
import functools

import jax
import jax.numpy as jnp
from jax import lax
from jax.experimental import pallas as pl
from jax.experimental.pallas import tpu as pltpu
from jax.experimental.pallas import tpu_sc as plsc

SEQ, B, EMBED, HEADS, HDIM = 2048, 1, 1024, 16, 64
E, K, FF = 8, 2, 2048
N = SEQ * B

BM = 256
BQ = 256
NPAD = K * N + E * BM
NB = NPAD // BM

_f32 = jnp.float32
_bf16 = jnp.bfloat16


def _ln(x, g, b):
    m = jnp.mean(x, axis=-1, keepdims=True)
    v = jnp.mean((x - m) ** 2, axis=-1, keepdims=True)
    return (x - m) * lax.rsqrt(v + 1e-5) * g + b


def _qkv_body(x_ref, sa_ref, g_ref, b_ref, wq_ref, bq_ref, wk_ref, bk_ref,
              wv_ref, bv_ref, q_ref, k_ref, v_ref):
    xq = _ln(x_ref[...], g_ref[...], b_ref[...]).astype(_bf16)
    xkv = _ln(sa_ref[...], g_ref[...], b_ref[...]).astype(_bf16)
    q_ref[...] = (jnp.dot(xq, wq_ref[...], preferred_element_type=_f32)
                  + bq_ref[...]).astype(_bf16)
    k_ref[...] = (jnp.dot(xkv, wk_ref[...], preferred_element_type=_f32)
                  + bk_ref[...]).astype(_bf16)
    v_ref[...] = (jnp.dot(xkv, wv_ref[...], preferred_element_type=_f32)
                  + bv_ref[...]).astype(_bf16)


def _attn_body(q_ref, k_ref, v_ref, pm_ref, o_ref):
    q = q_ref[0]
    k = k_ref[0]
    v = v_ref[0]
    s = lax.dot_general(q, k, (((1,), (1,)), ((), ())),
                        preferred_element_type=_f32) * 0.125
    s = s + pm_ref[...]
    m = jnp.max(s, axis=-1, keepdims=True)
    p = jnp.exp(s - m)
    o = jnp.dot(p.astype(_bf16), v, preferred_element_type=_f32)
    o_ref[0] = (o / jnp.sum(p, axis=-1, keepdims=True)).astype(_bf16)


def _oproj_router_body(a_ref, wo_ref, bo_ref, hm_ref, x_ref, g2_ref, b2_ref,
                       wg_ref, x1_ref, xn_ref, i1_ref, i2_ref, g1_ref, gg2_ref):
    y = jnp.dot(a_ref[...], wo_ref[...], preferred_element_type=_f32) + bo_ref[...]
    hm = hm_ref[...]
    x1 = x_ref[...] + y * hm
    x1_ref[...] = x1
    xn = _ln(x1, g2_ref[...], b2_ref[...])
    xn_ref[...] = xn
    lg = jnp.dot(xn, wg_ref[...], preferred_element_type=_f32)
    idx = lax.broadcasted_iota(jnp.int32, lg.shape, 1)
    m1 = jnp.max(lg, axis=-1, keepdims=True)
    i1 = jnp.min(jnp.where(lg == m1, idx, E), axis=-1, keepdims=True)
    lg2 = jnp.where(idx == i1, -jnp.inf, lg)
    m2 = jnp.max(lg2, axis=-1, keepdims=True)
    i2 = jnp.min(jnp.where(lg2 == m2, idx, E), axis=-1, keepdims=True)
    g1 = 1.0 / (1.0 + jnp.exp(m2 - m1))
    i1_ref[...] = i1
    i2_ref[...] = i2
    g1_ref[...] = g1 * hm
    gg2_ref[...] = (1.0 - g1) * hm


def _moe_body(eid_ref, xs_ref, w1_ref, b1_ref, w2_ref, b2_ref, ys_ref):
    bidx = pl.program_id(0)

    @pl.when(eid_ref[bidx] < E)
    def _():
        h = jnp.maximum(
            jnp.dot(xs_ref[...].astype(_bf16), w1_ref[0].astype(_bf16),
                    preferred_element_type=_f32) + b1_ref[0], 0.0)
        ys_ref[...] = jnp.dot(h.astype(_bf16), w2_ref[0].astype(_bf16),
                              preferred_element_type=_f32) + b2_ref[0]


def _fin_body(x_ref, y1_ref, y2_ref, g1_ref, g2_ref, o_ref):
    o_ref[...] = (x_ref[...] + g1_ref[...] * y1_ref[...]
                  + g2_ref[...] * y2_ref[...])


_SC_CORES, _SC_SUBCORES = 2, 16
_NW = _SC_CORES * _SC_SUBCORES
_CH = 32


def _sc_dispatch(xn, tok, pos):
    M = tok.shape[0]
    mpw = M // _NW
    nch = mpw // _CH
    mesh = plsc.VectorSubcoreMesh(core_axis_name="c", subcore_axis_name="s")

    @functools.partial(
        pl.kernel, mesh=mesh,
        out_type=jax.ShapeDtypeStruct((NPAD, EMBED), _f32),
        scratch_types=[
            pltpu.VMEM((2, _CH), jnp.int32),
            pltpu.VMEM((2, _CH), jnp.int32),
            pltpu.VMEM((2, _CH, EMBED), _f32),
            pltpu.SemaphoreType.DMA,
            pltpu.SemaphoreType.DMA,
            pltpu.SemaphoreType.DMA,
        ],
    )
    def k(xn_hbm, tok_hbm, pos_hbm, out_hbm, tok_v, pos_v, rows_v,
          gsem, ssem0, ssem1):
        wid = lax.axis_index("s") * _SC_CORES + lax.axis_index("c")
        base = wid * mpw
        ssems = (ssem0, ssem1)
        pend = [None, None]
        for c in range(nch):
            b = c % 2
            o = base + c * _CH
            if pend[b] is not None:
                pend[b].wait()
            pltpu.sync_copy(tok_hbm.at[pl.ds(o, _CH)], tok_v.at[b])
            pltpu.sync_copy(pos_hbm.at[pl.ds(o, _CH)], pos_v.at[b])
            pltpu.async_copy(xn_hbm.at[tok_v.at[b]], rows_v.at[b], gsem).wait()
            pend[b] = pltpu.async_copy(rows_v.at[b], out_hbm.at[pos_v.at[b]],
                                       ssems[b])
        for p in pend:
            if p is not None:
                p.wait()

    return k(xn, tok, pos)


def _sc_gather(table, idx):
    rows = idx.shape[0]
    rpw = rows // _NW
    nch = rpw // _CH
    mesh = plsc.VectorSubcoreMesh(core_axis_name="c", subcore_axis_name="s")

    @functools.partial(
        pl.kernel, mesh=mesh,
        out_type=jax.ShapeDtypeStruct((rows, table.shape[1]), _f32),
        scratch_types=[
            pltpu.VMEM((2, _CH), jnp.int32),
            pltpu.VMEM((2, _CH, table.shape[1]), _f32),
            pltpu.SemaphoreType.DMA,
            pltpu.SemaphoreType.DMA,
            pltpu.SemaphoreType.DMA,
        ],
    )
    def k(table_hbm, idx_hbm, out_hbm, idx_v, rows_v, gsem, ssem0, ssem1):
        wid = lax.axis_index("s") * _SC_CORES + lax.axis_index("c")
        base = wid * rpw
        ssems = (ssem0, ssem1)
        pend = [None, None]
        for c in range(nch):
            b = c % 2
            o = base + c * _CH
            if pend[b] is not None:
                pend[b].wait()
            pltpu.sync_copy(idx_hbm.at[pl.ds(o, _CH)], idx_v.at[b])
            pltpu.async_copy(table_hbm.at[idx_v.at[b]], rows_v.at[b],
                             gsem).wait()
            pend[b] = pltpu.async_copy(rows_v.at[b],
                                       out_hbm.at[pl.ds(o, _CH)], ssems[b])
        for p in pend:
            if p is not None:
                p.wait()

    return k(table, idx)


def _route_metadata(i1, i2):
    ei = jnp.stack([i1, i2], axis=1).reshape(-1)
    tok = (jnp.arange(K * N, dtype=jnp.int32) // K)
    oh = (ei[:, None] == jnp.arange(E, dtype=jnp.int32)[None, :]).astype(jnp.int32)
    counts = jnp.sum(oh, axis=0)
    padded = ((counts + BM - 1) // BM) * BM
    ends = jnp.cumsum(padded)
    offsets = ends - padded
    csum = jnp.cumsum(oh, axis=0)
    rankj = jnp.sum((csum - 1) * oh, axis=1)
    posj = jnp.sum(offsets[None, :] * oh, axis=1) + rankj
    posj = posj.astype(jnp.int32)
    bstart = jnp.arange(NB, dtype=jnp.int32) * BM
    beid = jnp.sum((bstart[:, None] >= ends[None, :]).astype(jnp.int32), axis=1)
    beid = jnp.where(bstart < ends[-1], beid, E).astype(jnp.int32)
    return tok, posj, beid


def kernel(x, self_attn_input, halt_mask, layer_idx, encoder_padding_mask,
           ln1_g, ln1_b, ln2_g, ln2_b, Wq, bq, Wk, bk, Wv, bv, Wo, bo,
           Wg, W1, b1, W2, b2):
    x2 = x.reshape(N, EMBED)
    sa2 = self_attn_input.reshape(N, EMBED)
    hm = (~halt_mask.reshape(-1))[:, None].astype(_f32)
    pm = jnp.where(encoder_padding_mask.reshape(1, -1), -1e8, 0.0).astype(_f32)

    g1v = ln1_g.reshape(1, EMBED)
    b1v = ln1_b.reshape(1, EMBED)
    g2v = ln2_g.reshape(1, EMBED)
    b2v = ln2_b.reshape(1, EMBED)
    bqv = bq.reshape(1, EMBED)
    bkv = bk.reshape(1, EMBED)
    bvv = bv.reshape(1, EMBED)
    bov = bo.reshape(1, EMBED)

    row_spec = pl.BlockSpec((BM, EMBED), lambda i: (i, 0))
    w_spec = pl.BlockSpec((EMBED, EMBED), lambda i: (0, 0))
    vec_spec = pl.BlockSpec((1, EMBED), lambda i: (0, 0))
    col1_spec = pl.BlockSpec((BM, 1), lambda i: (i, 0))

    q, kk, v = pl.pallas_call(
        _qkv_body,
        grid=(N // BM,),
        in_specs=[row_spec, row_spec, vec_spec, vec_spec,
                  w_spec, vec_spec, w_spec, vec_spec, w_spec, vec_spec],
        out_specs=[row_spec, row_spec, row_spec],
        out_shape=[jax.ShapeDtypeStruct((N, EMBED), _bf16)] * 3,
    )(x2, sa2, g1v, b1v, Wq.astype(_bf16), bqv, Wk.astype(_bf16), bkv,
      Wv.astype(_bf16), bvv)

    q3 = q.reshape(N, HEADS, HDIM).transpose(1, 0, 2)
    k3 = kk.reshape(N, HEADS, HDIM).transpose(1, 0, 2)
    v3 = v.reshape(N, HEADS, HDIM).transpose(1, 0, 2)

    qb_spec = pl.BlockSpec((1, BQ, HDIM), lambda h, i: (h, i, 0))
    kv_spec = pl.BlockSpec((1, N, HDIM), lambda h, i: (h, 0, 0))
    pm_spec = pl.BlockSpec((1, N), lambda h, i: (0, 0))
    attn3 = pl.pallas_call(
        _attn_body,
        grid=(HEADS, N // BQ),
        in_specs=[qb_spec, kv_spec, kv_spec, pm_spec],
        out_specs=qb_spec,
        out_shape=jax.ShapeDtypeStruct((HEADS, N, HDIM), _bf16),
    )(q3, k3, v3, pm)
    attn = attn3.transpose(1, 0, 2).reshape(N, EMBED)

    wg_spec = pl.BlockSpec((EMBED, E), lambda i: (0, 0))
    i1_spec = pl.BlockSpec((BM, 1), lambda i: (i, 0))
    x1, xn, i1, i2, gg1, gg2 = pl.pallas_call(
        _oproj_router_body,
        grid=(N // BM,),
        in_specs=[row_spec, w_spec, vec_spec, col1_spec, row_spec,
                  vec_spec, vec_spec, wg_spec],
        out_specs=[row_spec, row_spec, i1_spec, i1_spec, i1_spec, i1_spec],
        out_shape=[
            jax.ShapeDtypeStruct((N, EMBED), _f32),
            jax.ShapeDtypeStruct((N, EMBED), _f32),
            jax.ShapeDtypeStruct((N, 1), jnp.int32),
            jax.ShapeDtypeStruct((N, 1), jnp.int32),
            jax.ShapeDtypeStruct((N, 1), _f32),
            jax.ShapeDtypeStruct((N, 1), _f32),
        ],
    )(attn, Wo.astype(_bf16), bov, hm, x2, g2v, b2v, Wg)

    tok, pos, beid = _route_metadata(i1.reshape(-1), i2.reshape(-1))

    xs = _sc_dispatch(xn, tok, pos)

    def _wsel(b, eid):
        return jnp.minimum(eid[b], E - 1)

    grid_spec = pltpu.PrefetchScalarGridSpec(
        num_scalar_prefetch=1,
        grid=(NB,),
        in_specs=[
            pl.BlockSpec((BM, EMBED), lambda b, eid: (b, 0)),
            pl.BlockSpec((1, EMBED, FF), lambda b, eid: (_wsel(b, eid), 0, 0)),
            pl.BlockSpec((1, 1, FF), lambda b, eid: (_wsel(b, eid), 0, 0)),
            pl.BlockSpec((1, FF, EMBED), lambda b, eid: (_wsel(b, eid), 0, 0)),
            pl.BlockSpec((1, 1, EMBED), lambda b, eid: (_wsel(b, eid), 0, 0)),
        ],
        out_specs=pl.BlockSpec((BM, EMBED), lambda b, eid: (b, 0)),
    )
    ys = pl.pallas_call(
        _moe_body,
        grid_spec=grid_spec,
        out_shape=jax.ShapeDtypeStruct((NPAD, EMBED), _f32),
    )(beid, xs, W1, b1.reshape(E, 1, FF), W2, b2.reshape(E, 1, EMBED))

    pos2 = pos.reshape(N, K)
    pos_all = pos2.T.reshape(-1)
    y12 = _sc_gather(ys, pos_all)
    y1 = y12[:N]
    y2 = y12[N:]

    out = pl.pallas_call(
        _fin_body,
        grid=(N // BM,),
        in_specs=[row_spec, row_spec, row_spec, col1_spec, col1_spec],
        out_specs=row_spec,
        out_shape=jax.ShapeDtypeStruct((N, EMBED), _f32),
    )(x1, y1, y2, gg1, gg2)

    return out.reshape(SEQ, B, EMBED), jnp.float32(0.0)
